# baseline (speedup 1.0000x reference)
import jax
import jax.numpy as jnp
from jax.experimental import pallas as pl
from jax.experimental.pallas import tpu as pltpu

_B, _D, _E = 8192, 4096, 64
_H1, _H2, _H3 = 128, 256, 128
_R = 1024
_N = _B // _R

_DN = (((1,), (1,)), ((), ()))


def _top2(logits, rows):
    col = jax.lax.broadcasted_iota(jnp.int32, (rows, _E), 1)
    m1 = jnp.max(logits, axis=1, keepdims=True)
    i1 = jnp.min(jnp.where(logits == m1, col, _E), axis=1, keepdims=True)
    masked = jnp.where(col == i1, -jnp.inf, logits)
    m2 = jnp.max(masked, axis=1, keepdims=True)
    i2 = jnp.min(jnp.where(masked == m2, col, _E), axis=1, keepdims=True)
    return m1, i1, m2, i2


def _gating_kernel(x_ref, w1_ref, b1_ref, w2_ref, b2_ref, w3_ref, b3_ref,
                   w4_ref, b4_ref, w1b_ref, w2b_ref, w3b_ref, w4b_ref,
                   out_ref, acc_ref):
    step = pl.program_id(0)

    h = jax.lax.dot_general(x_ref[...].astype(jnp.bfloat16), w1b_ref[...],
                            _DN, preferred_element_type=jnp.float32) + b1_ref[...]
    h = jnp.maximum(h, 0.0)
    h = jax.lax.dot_general(h.astype(jnp.bfloat16), w2b_ref[...], _DN,
                            preferred_element_type=jnp.float32) + b2_ref[...]
    h = jnp.where(h >= 0, h, 0.01 * h)
    h = jax.lax.dot_general(h.astype(jnp.bfloat16), w3b_ref[...], _DN,
                            preferred_element_type=jnp.float32) + b3_ref[...]
    h = jnp.where(h >= 0, h, 0.01 * h)
    logits = jax.lax.dot_general(h.astype(jnp.bfloat16), w4b_ref[...], _DN,
                                 preferred_element_type=jnp.float32) + b4_ref[...]

    m1, _, m2, _ = _top2(logits, _R)

    psum = jnp.sum(m1) + jnp.sum(m2)
    prev = jnp.where(step == 0, 0.0, acc_ref[0])
    total = prev + psum
    acc_ref[0] = total

    out_ref[...] = jnp.zeros((_R, _E), jnp.float32)

    @pl.when(step == _N - 1)
    def _write_row0():
        xr = x_ref[0:8, :]
        hr = jax.lax.dot_general(xr, w1_ref[...], _DN,
                                 preferred_element_type=jnp.float32) + b1_ref[...]
        hr = jnp.maximum(hr, 0.0)
        hr = jax.lax.dot_general(hr, w2_ref[...], _DN,
                                 preferred_element_type=jnp.float32) + b2_ref[...]
        hr = jnp.where(hr >= 0, hr, 0.01 * hr)
        hr = jax.lax.dot_general(hr, w3_ref[...], _DN,
                                 preferred_element_type=jnp.float32) + b3_ref[...]
        hr = jnp.where(hr >= 0, hr, 0.01 * hr)
        lr = jax.lax.dot_general(hr, w4_ref[...], _DN,
                                 preferred_element_type=jnp.float32) + b4_ref[...]
        v1, j1, v2, j2 = _top2(lr[0:1, :], 1)
        lane = jax.lax.broadcasted_iota(jnp.int32, (1, _E), 1)
        row = (jnp.where(lane == j1, v1 / total, 0.0)
               + jnp.where(lane == j2, v2 / total, 0.0))
        out_ref[0:1, :] = row


def kernel(x, W1, b1, W2, b2, W3, b3, W4, b4):
    b1r = b1.reshape(1, _H1)
    b2r = b2.reshape(1, _H2)
    b3r = b3.reshape(1, _H3)
    b4r = b4.reshape(1, _E)
    rev = lambda i: (_N - 1 - i, 0)
    fixed = lambda i: (0, 0)
    return pl.pallas_call(
        _gating_kernel,
        grid=(_N,),
        in_specs=[
            pl.BlockSpec((_R, _D), rev),
            pl.BlockSpec((_H1, _D), fixed),
            pl.BlockSpec((1, _H1), fixed),
            pl.BlockSpec((_H2, _H1), fixed),
            pl.BlockSpec((1, _H2), fixed),
            pl.BlockSpec((_H3, _H2), fixed),
            pl.BlockSpec((1, _H3), fixed),
            pl.BlockSpec((_E, _H3), fixed),
            pl.BlockSpec((1, _E), fixed),
            pl.BlockSpec((_H1, _D), fixed),
            pl.BlockSpec((_H2, _H1), fixed),
            pl.BlockSpec((_H3, _H2), fixed),
            pl.BlockSpec((_E, _H3), fixed),
        ],
        out_specs=pl.BlockSpec((_R, _E), rev),
        out_shape=jax.ShapeDtypeStruct((_B, _E), jnp.float32),
        scratch_shapes=[pltpu.SMEM((1,), jnp.float32)],
    )(x, W1, b1r, W2, b2r, W3, b3r, W4, b4r,
      W1.astype(jnp.bfloat16), W2.astype(jnp.bfloat16),
      W3.astype(jnp.bfloat16), W4.astype(jnp.bfloat16))


# speedup vs baseline: 1.1692x; 1.1692x over previous
import jax
import jax.numpy as jnp
from jax.experimental import pallas as pl
from jax.experimental.pallas import tpu as pltpu

_B, _D, _E = 8192, 4096, 64
_H1, _H2, _H3 = 128, 256, 128
_R = 1024
_N = _B // _R

_DN = (((1,), (1,)), ((), ()))


def _gating_kernel(x_ref, w1_ref, b1_ref, w2_ref, b2_ref, w3_ref, b3_ref,
                   w4_ref, b4_ref, out_ref, acc_ref):
    step = pl.program_id(0)

    x = x_ref[...]
    h = jax.lax.dot_general(x, w1_ref[...], _DN,
                            preferred_element_type=jnp.float32) + b1_ref[...]
    h = jnp.maximum(h, 0.0)
    h = jax.lax.dot_general(h, w2_ref[...], _DN,
                            preferred_element_type=jnp.float32) + b2_ref[...]
    h = jnp.where(h >= 0, h, 0.01 * h)
    h = jax.lax.dot_general(h, w3_ref[...], _DN,
                            preferred_element_type=jnp.float32) + b3_ref[...]
    h = jnp.where(h >= 0, h, 0.01 * h)
    logits = jax.lax.dot_general(h, w4_ref[...], _DN,
                                 preferred_element_type=jnp.float32) + b4_ref[...]

    col = jax.lax.broadcasted_iota(jnp.int32, (_R, _E), 1)
    m1 = jnp.max(logits, axis=1, keepdims=True)
    i1 = jnp.min(jnp.where(logits == m1, col, _E), axis=1, keepdims=True)
    masked = jnp.where(col == i1, -jnp.inf, logits)
    m2 = jnp.max(masked, axis=1, keepdims=True)
    i2 = jnp.min(jnp.where(masked == m2, col, _E), axis=1, keepdims=True)

    psum = jnp.sum(m1) + jnp.sum(m2)
    prev = jnp.where(step == 0, 0.0, acc_ref[0])
    total = prev + psum
    acc_ref[0] = total

    out_ref[...] = jnp.zeros((_R, _E), jnp.float32)

    @pl.when(step == _N - 1)
    def _write_row0():
        lane = jax.lax.broadcasted_iota(jnp.int32, (1, _E), 1)
        row = (jnp.where(lane == i1[0:1], m1[0:1] / total, 0.0)
               + jnp.where(lane == i2[0:1], m2[0:1] / total, 0.0))
        out_ref[0:1, :] = row


def kernel(x, W1, b1, W2, b2, W3, b3, W4, b4):
    b1r = b1.reshape(1, _H1)
    b2r = b2.reshape(1, _H2)
    b3r = b3.reshape(1, _H3)
    b4r = b4.reshape(1, _E)
    rev = lambda i: (_N - 1 - i, 0)
    fixed = lambda i: (0, 0)
    return pl.pallas_call(
        _gating_kernel,
        grid=(_N,),
        in_specs=[
            pl.BlockSpec((_R, _D), rev),
            pl.BlockSpec((_H1, _D), fixed),
            pl.BlockSpec((1, _H1), fixed),
            pl.BlockSpec((_H2, _H1), fixed),
            pl.BlockSpec((1, _H2), fixed),
            pl.BlockSpec((_H3, _H2), fixed),
            pl.BlockSpec((1, _H3), fixed),
            pl.BlockSpec((_E, _H3), fixed),
            pl.BlockSpec((1, _E), fixed),
        ],
        out_specs=pl.BlockSpec((_R, _E), rev),
        out_shape=jax.ShapeDtypeStruct((_B, _E), jnp.float32),
        scratch_shapes=[pltpu.SMEM((1,), jnp.float32)],
    )(x, W1, b1r, W2, b2r, W3, b3r, W4, b4r)
